# Initial kernel scaffold; baseline (speedup 1.0000x reference)
import jax
import jax.numpy as jnp
from jax.experimental import pallas as pl


def kernel(z, embedding_weight):
    raise NotImplementedError("write your pallas kernel here")


# speedup vs baseline: 1.4671x; 1.4671x over previous
import jax
import jax.numpy as jnp
from jax.experimental import pallas as pl
from jax.experimental.pallas import tpu as pltpu

_N_E = 1024
_D = 64
_BETA = 0.25
_TN = 256


def _rq_body(r_ref, e_ref, et_ref, idx_ref, q_ref, sums_ref):
    g = pl.program_id(0)
    r = r_ref[...]
    e = e_ref[...]
    et = et_ref[...]
    b2_row = jnp.sum(et * et, axis=0, keepdims=True)
    a2_col = jnp.sum(r * r, axis=1, keepdims=True)

    s = jax.lax.dot_general(r, et, (((1,), (0,)), ((), ())),
                            preferred_element_type=jnp.float32)
    d = (a2_col - 2.0 * s) + b2_row
    vd = jnp.min(d, axis=1, keepdims=True)
    col = jax.lax.broadcasted_iota(jnp.int32, d.shape, 1)
    idx2d = jnp.min(jnp.where(d == vd, col, _N_E), axis=1, keepdims=True)
    idx_ref[...] = idx2d

    onehot = (col == idx2d).astype(jnp.float32)
    q_ref[...] = jax.lax.dot_general(onehot, e, (((1,), (0,)), ((), ())),
                                     preferred_element_type=jnp.float32)

    counts = jnp.sum(onehot, axis=0, keepdims=True)
    sum_bb = jnp.sum(counts * b2_row, axis=1, keepdims=True)[0, 0]
    sum_vd = jnp.sum(vd, axis=0, keepdims=True)[0, 0]
    sum_z2 = jnp.sum(a2_col, axis=0, keepdims=True)[0, 0]
    sum_qz = 0.5 * (sum_z2 + sum_bb - sum_vd)

    @pl.when(g == 0)
    def _init():
        sums_ref[0] = sum_bb
        sums_ref[1] = sum_qz
        sums_ref[2] = sum_z2

    @pl.when(g != 0)
    def _acc():
        sums_ref[0] += sum_bb
        sums_ref[1] += sum_qz
        sums_ref[2] += sum_z2


def _rq_call(r, e, et, interpret=False):
    n = r.shape[0]
    grid = n // _TN
    return pl.pallas_call(
        _rq_body,
        grid=(grid,),
        in_specs=[
            pl.BlockSpec((_TN, _D), lambda g: (g, 0)),
            pl.BlockSpec((_N_E, _D), lambda g: (0, 0)),
            pl.BlockSpec((_D, _N_E), lambda g: (0, 0)),
        ],
        out_specs=[
            pl.BlockSpec((_TN, 1), lambda g: (g, 0)),
            pl.BlockSpec((_TN, _D), lambda g: (g, 0)),
            pl.BlockSpec(memory_space=pltpu.SMEM),
        ],
        out_shape=[
            jax.ShapeDtypeStruct((n, 1), jnp.int32),
            jax.ShapeDtypeStruct((n, _D), jnp.float32),
            jax.ShapeDtypeStruct((3,), jnp.float32),
        ],
        interpret=interpret,
    )(r, e, et)


def kernel(z, embedding_weight):
    z = z.astype(jnp.float32)
    B, C, H, W = z.shape
    r = jnp.transpose(z, (0, 2, 3, 1)).reshape(-1, C)
    et = jnp.transpose(embedding_weight, (1, 0))
    idx, q, sums = _rq_call(r, embedding_weight, et)

    z_hat = jnp.transpose((4.0 * q).reshape(B, H, W, C), (0, 3, 1, 2))
    m = jnp.float32(B * C * H * W)
    loss = ((1.0 + _BETA) / 4.0) * (30.0 * sums[0] - 20.0 * sums[1]
                                    + 4.0 * sums[2]) / m
    idx3 = idx.reshape(B, W, W)
    total_idx = jnp.concatenate([idx3, idx3, idx3, idx3], axis=1)
    return (z_hat, loss, total_idx)
